# baseline (speedup 1.0000x reference)
import functools

import jax
import jax.numpy as jnp
from jax import lax
from jax.experimental import pallas as pl
from jax.experimental.pallas import tpu as pltpu
from jax.experimental.pallas import tpu_sc as plsc

NC, NS, L = 2, 16, 16
NW = NC * NS
G = 128
G2 = G * G
G3 = G * G * G
C = 4
ROW = 16

WBLK = 32
INW = WBLK + 1
INBUF = C * INW * G + 128

N_PTS = 1048576
PTS_PER_W = N_PTS // NW
BATCH = 1024
NBATCH = PTS_PER_W // BATCH
GROUPS = BATCH // L
NSTREAM = 2 * BATCH // 128

_mesh = plsc.VectorSubcoreMesh(core_axis_name="c", subcore_axis_name="s")
_params = pltpu.CompilerParams(
    needs_layout_passes=False, use_tc_tiling_on_sc=False
)


@functools.partial(
    pl.kernel,
    out_type=jax.ShapeDtypeStruct((G3 * ROW,), jnp.float32),
    mesh=_mesh,
    scratch_types=[
        pltpu.VMEM((INBUF,), jnp.float32),
        pltpu.VMEM((WBLK * G * ROW,), jnp.float32),
    ],
    compiler_params=_params,
)
def _build(field_hbm, table_hbm, inbuf, outbuf):
    wid = lax.axis_index("s") * NC + lax.axis_index("c")
    j = lax.iota(jnp.int32, L)
    offv = (j & 3) * (INW * G) + ((j >> 2) & 1) + (j >> 3) * G

    def per_h(t, _):
        h = wid * (G // NW) + t
        for wb in range(4):
            nwords = (INW if wb < 3 else WBLK) * G
            for c in range(C):
                src = c * G3 + h * G2 + wb * WBLK * G
                pltpu.sync_copy(
                    field_hbm.at[pl.ds(src, nwords)],
                    inbuf.at[pl.ds(c * INW * G, nwords)],
                )

            def per_row(row, _):
                v = plsc.load_gather(inbuf, [row + offv])
                outbuf[pl.ds(row * ROW, ROW)] = v
                return ()

            lax.fori_loop(0, WBLK * G, per_row, (), unroll=4)
            dst = h * G2 * ROW + wb * WBLK * G * ROW
            pltpu.sync_copy(outbuf, table_hbm.at[pl.ds(dst, WBLK * G * ROW)])
        return ()

    lax.fori_loop(0, G // NW, per_h, ())


@functools.partial(
    pl.kernel,
    out_type=jax.ShapeDtypeStruct((N_PTS * C,), jnp.float32),
    mesh=_mesh,
    scratch_types=[
        pltpu.VMEM((3 * BATCH,), jnp.float32),
        pltpu.VMEM((3 * BATCH,), jnp.float32),
        pltpu.VMEM((NSTREAM, 128), jnp.int32),
        pltpu.VMEM((2 * BATCH, ROW), jnp.float32),
        pltpu.VMEM((C * BATCH,), jnp.float32),
        pltpu.VMEM((L,), jnp.float32),
        pltpu.SemaphoreType.DMA,
    ],
    compiler_params=_params,
)
def _sample(x_hbm, scale_hbm, table_hbm, out_hbm,
            xbuf, fracbuf, idxbuf, gathbuf, outbuf, scalebuf, gsem):
    wid = lax.axis_index("s") * NC + lax.axis_index("c")
    base_pt = wid * PTS_PER_W
    iv = lax.iota(jnp.int32, L)
    pltpu.sync_copy(scale_hbm, scalebuf)

    def per_batch(b, _):
        pt0 = base_pt + b * BATCH
        pltpu.sync_copy(x_hbm.at[pl.ds(pt0 * 3, 3 * BATCH)], xbuf)
        scale = scalebuf[...]

        def stage_a(g, _):
            xb = 48 * g + 3 * iv
            xh = plsc.load_gather(xbuf, [xb])
            xw = plsc.load_gather(xbuf, [xb + 1])
            xd = plsc.load_gather(xbuf, [xb + 2])

            def dim(xc):
                f = jnp.clip(xc * scale + 63.5, 0.0, 127.0)
                i0 = jnp.minimum(f.astype(jnp.int32), G - 2)
                return i0, f - i0.astype(jnp.float32)

            h0, fh = dim(xh)
            w0, fw = dim(xw)
            d0, fd = dim(xd)
            fracbuf[pl.ds(g * L, L)] = fh
            fracbuf[pl.ds(BATCH + g * L, L)] = fw
            fracbuf[pl.ds(2 * BATCH + g * L, L)] = fd
            r0 = h0 * G2 + w0 * G + d0
            row = g // 8
            col = (g % 8) * L
            idxbuf[row, pl.ds(col, L)] = r0
            idxbuf[row + NSTREAM // 2, pl.ds(col, L)] = r0 + G2
            return ()

        lax.fori_loop(0, GROUPS, stage_a, ())

        descs = [
            pltpu.make_async_copy(
                table_hbm.at[idxbuf.at[k]],
                gathbuf.at[pl.ds(k * 128, 128)],
                gsem,
            )
            for k in range(NSTREAM)
        ]
        for d in descs:
            d.start()
        for d in descs:
            d.wait()

        def stage_b(g, _):
            fh = fracbuf[pl.ds(g * L, L)]
            fw = fracbuf[pl.ds(BATCH + g * L, L)]
            fd = fracbuf[pl.ds(2 * BATCH + g * L, L)]
            rows = g * L + iv

            def corner(hh, ww, dd, c):
                return plsc.load_gather(
                    gathbuf,
                    [rows + hh * BATCH, jnp.full((L,), ww * 8 + dd * 4 + c,
                                                 jnp.int32)],
                )

            for c in range(C):
                v000 = corner(0, 0, 0, c)
                v001 = corner(0, 0, 1, c)
                v010 = corner(0, 1, 0, c)
                v011 = corner(0, 1, 1, c)
                v100 = corner(1, 0, 0, c)
                v101 = corner(1, 0, 1, c)
                v110 = corner(1, 1, 0, c)
                v111 = corner(1, 1, 1, c)
                a00 = v000 + fd * (v001 - v000)
                a01 = v010 + fd * (v011 - v010)
                a10 = v100 + fd * (v101 - v100)
                a11 = v110 + fd * (v111 - v110)
                b0 = a00 + fw * (a01 - a00)
                b1 = a10 + fw * (a11 - a10)
                res = b0 + fh * (b1 - b0)
                plsc.store_scatter(outbuf, [4 * rows + c], res)
            return ()

        lax.fori_loop(0, GROUPS, stage_b, ())
        pltpu.sync_copy(outbuf, out_hbm.at[pl.ds(pt0 * C, C * BATCH)])
        return ()

    lax.fori_loop(0, NBATCH, per_batch, ())


def kernel(x, field, extent):
    scale = jnp.full((L,), 127.0, jnp.float32) / extent.astype(jnp.float32)
    table = _build(field.reshape(-1))
    out = _sample(x.reshape(-1), scale, table.reshape(G3, ROW))
    return out.reshape(N_PTS, C)


# speedup vs baseline: 2.5428x; 2.5428x over previous
import functools

import jax
import jax.numpy as jnp
from jax import lax
from jax.experimental import pallas as pl
from jax.experimental.pallas import tpu as pltpu
from jax.experimental.pallas import tpu_sc as plsc

NC, NS, L = 2, 16, 16
NW = NC * NS
G = 128
G2 = G * G
G3 = G * G * G
C = 4
ROW = 16

WBLK = 32
INW = WBLK + 1
INBUF = C * INW * G + 128

N_PTS = 1048576
PTS_PER_W = N_PTS // NW
BATCH = 1024
NBATCH = PTS_PER_W // BATCH
GROUPS = BATCH // L
NSTREAM = 2 * BATCH // 128

_mesh = plsc.VectorSubcoreMesh(core_axis_name="c", subcore_axis_name="s")
_params = pltpu.CompilerParams(
    needs_layout_passes=False, use_tc_tiling_on_sc=False
)


@functools.partial(
    pl.kernel,
    out_type=jax.ShapeDtypeStruct((G3 * ROW,), jnp.float32),
    mesh=_mesh,
    scratch_types=[
        pltpu.VMEM((INBUF,), jnp.float32),
        pltpu.VMEM((WBLK * G * ROW,), jnp.float32),
    ],
    compiler_params=_params,
)
def _build(field_hbm, table_hbm, inbuf, outbuf):
    wid = lax.axis_index("s") * NC + lax.axis_index("c")
    j = lax.iota(jnp.int32, L)
    offv = (j & 3) * (INW * G) + ((j >> 2) & 1) + (j >> 3) * G

    def per_h(t, _):
        h = wid * (G // NW) + t
        for wb in range(4):
            nwords = (INW if wb < 3 else WBLK) * G
            for c in range(C):
                src = c * G3 + h * G2 + wb * WBLK * G
                pltpu.sync_copy(
                    field_hbm.at[pl.ds(src, nwords)],
                    inbuf.at[pl.ds(c * INW * G, nwords)],
                )

            def per_row(row, _):
                v = plsc.load_gather(inbuf, [row + offv])
                outbuf[pl.ds(row * ROW, ROW)] = v
                return ()

            lax.fori_loop(0, WBLK * G, per_row, (), unroll=4)
            dst = h * G2 * ROW + wb * WBLK * G * ROW
            pltpu.sync_copy(outbuf, table_hbm.at[pl.ds(dst, WBLK * G * ROW)])
        return ()

    lax.fori_loop(0, G // NW, per_h, ())


@functools.partial(
    pl.kernel,
    out_type=jax.ShapeDtypeStruct((N_PTS * C,), jnp.float32),
    mesh=_mesh,
    scratch_types=[
        pltpu.VMEM((3 * BATCH,), jnp.float32),
        pltpu.VMEM((3 * BATCH,), jnp.float32),
        pltpu.VMEM((NSTREAM, 128), jnp.int32),
        pltpu.VMEM((2 * BATCH, ROW), jnp.float32),
        pltpu.VMEM((C * BATCH,), jnp.float32),
        pltpu.VMEM((L,), jnp.float32),
        pltpu.SemaphoreType.DMA,
    ],
    compiler_params=_params,
)
def _sample(x_hbm, scale_hbm, table_hbm, out_hbm,
            xbuf, fracbuf, idxbuf, gathbuf, outbuf, scalebuf, gsem):
    wid = lax.axis_index("s") * NC + lax.axis_index("c")
    base_pt = wid * PTS_PER_W
    iv = lax.iota(jnp.int32, L)
    pltpu.sync_copy(scale_hbm, scalebuf)

    def per_batch(b, _):
        pt0 = base_pt + b * BATCH
        for c in range(3):
            pltpu.sync_copy(
                x_hbm.at[pl.ds(c * N_PTS + pt0, BATCH)],
                xbuf.at[pl.ds(c * BATCH, BATCH)],
            )
        scale = scalebuf[...]

        def stage_a(g, _):
            xh = xbuf[pl.ds(g * L, L)]
            xw = xbuf[pl.ds(BATCH + g * L, L)]
            xd = xbuf[pl.ds(2 * BATCH + g * L, L)]

            def dim(xc):
                f = jnp.clip(xc * scale + 63.5, 0.0, 127.0)
                i0 = jnp.minimum(f.astype(jnp.int32), G - 2)
                return i0, f - i0.astype(jnp.float32)

            h0, fh = dim(xh)
            w0, fw = dim(xw)
            d0, fd = dim(xd)
            fracbuf[pl.ds(g * L, L)] = fh
            fracbuf[pl.ds(BATCH + g * L, L)] = fw
            fracbuf[pl.ds(2 * BATCH + g * L, L)] = fd
            r0 = h0 * G2 + w0 * G + d0
            row = g // 8
            col = (g % 8) * L
            idxbuf[row, pl.ds(col, L)] = r0
            idxbuf[row + NSTREAM // 2, pl.ds(col, L)] = r0 + G2
            return ()

        lax.fori_loop(0, GROUPS, stage_a, ())

        descs = [
            pltpu.make_async_copy(
                table_hbm.at[idxbuf.at[k]],
                gathbuf.at[pl.ds(k * 128, 128)],
                gsem,
            )
            for k in range(NSTREAM)
        ]
        for d in descs:
            d.start()
        for d in descs:
            d.wait()

        def stage_b(g, _):
            fh = fracbuf[pl.ds(g * L, L)]
            fw = fracbuf[pl.ds(BATCH + g * L, L)]
            fd = fracbuf[pl.ds(2 * BATCH + g * L, L)]
            rows = g * L + iv

            def corner(hh, ww, dd, c):
                return plsc.load_gather(
                    gathbuf,
                    [rows + hh * BATCH, jnp.full((L,), ww * 8 + dd * 4 + c,
                                                 jnp.int32)],
                )

            for c in range(C):
                v000 = corner(0, 0, 0, c)
                v001 = corner(0, 0, 1, c)
                v010 = corner(0, 1, 0, c)
                v011 = corner(0, 1, 1, c)
                v100 = corner(1, 0, 0, c)
                v101 = corner(1, 0, 1, c)
                v110 = corner(1, 1, 0, c)
                v111 = corner(1, 1, 1, c)
                a00 = v000 + fd * (v001 - v000)
                a01 = v010 + fd * (v011 - v010)
                a10 = v100 + fd * (v101 - v100)
                a11 = v110 + fd * (v111 - v110)
                b0 = a00 + fw * (a01 - a00)
                b1 = a10 + fw * (a11 - a10)
                res = b0 + fh * (b1 - b0)
                outbuf[pl.ds(c * BATCH + g * L, L)] = res
            return ()

        lax.fori_loop(0, GROUPS, stage_b, ())
        for c in range(C):
            pltpu.sync_copy(
                outbuf.at[pl.ds(c * BATCH, BATCH)],
                out_hbm.at[pl.ds(c * N_PTS + pt0, BATCH)],
            )
        return ()

    lax.fori_loop(0, NBATCH, per_batch, ())


def kernel(x, field, extent):
    scale = jnp.full((L,), 127.0, jnp.float32) / extent.astype(jnp.float32)
    table = _build(field.reshape(-1))
    out = _sample(x.T.reshape(-1), scale, table.reshape(G3, ROW))
    return out.reshape(C, N_PTS).T
